# baseline (speedup 1.0000x reference)
import functools
import math

import jax
import jax.numpy as jnp
import numpy as np
from jax.experimental import pallas as pl

T = 4096
HIDDEN = 4096
QR_RANK = 1536
H = 32
D = 128
ROPE = 64
RATIO = 4
BLOCKS = T // RATIO
TOPK = 512
EPS = 1e-6
SCALE = D ** -0.5
NEG = -1e30

TT = 512
TT3 = 256
RT = 256

bf16 = jnp.bfloat16
f32 = jnp.float32


def _swap_pairs(x):
    lane = jax.lax.broadcasted_iota(jnp.int32, x.shape, len(x.shape) - 1)
    even = (lane & 1) == 0
    return jnp.where(even, jnp.roll(x, -1, axis=-1), jnp.roll(x, 1, axis=-1))


def _k1_body(h_ref, wkv_ref, wgate_ref, ww_ref, ape_ref, kvape_ref, w_ref):
    hb = h_ref[...].astype(bf16)
    kv = jnp.dot(hb, wkv_ref[...].astype(bf16), preferred_element_type=f32)
    gate_lin = jnp.dot(hb, wgate_ref[...].astype(bf16),
                       preferred_element_type=f32)
    gate = 1.0 / (1.0 + jnp.exp(-gate_lin))
    kvape_ref[...] = kv * gate + ape_ref[...]
    w_ref[...] = jnp.dot(hb, ww_ref[...].astype(bf16),
                         preferred_element_type=f32)


def _k2_body(r0_ref, r1_ref, r2_ref, r3_ref, pre_ref):
    refs = (r0_ref, r1_ref, r2_ref, r3_ref)
    t = []
    for j in range(RATIO):
        rj = refs[j][...]
        a = jnp.roll(rj[:, :D], 1, axis=0)
        row = jax.lax.broadcasted_iota(jnp.int32, a.shape, 0)
        t.append(jnp.where(row == 0, 0.0, a))
    for j in range(RATIO):
        t.append(refs[j][...][:, D:])
    y = [t[j] + t[j + 4] for j in range(4)]
    z = [y[j] + y[j + 2] for j in range(2)]
    pre_ref[...] = (z[0] + z[1]) * (1.0 / (2 * RATIO))


def _k2b_body(pre_ref, rstd_ref, normw_ref, cosk_ref, sink_ref, kk_ref):
    kk = (pre_ref[...] * rstd_ref[...]) * normw_ref[...]
    kk_ref[...] = kk * cosk_ref[...] + _swap_pairs(kk) * sink_ref[...]


def _k3_body(qr_ref, wqb_ref, w_ref, kk_ref, cosq_ref, sinq_ref, pos_ref,
             out_ref):
    q = jnp.dot(qr_ref[...].astype(bf16), wqb_ref[...].astype(bf16),
                preferred_element_type=f32)
    cosq = cosq_ref[...]
    sinq = sinq_ref[...]
    kkb = kk_ref[...].astype(bf16)
    wb = w_ref[...].astype(bf16).astype(f32)

    def term(h):
        qh = q[:, h * D:(h + 1) * D]
        qh = qh * cosq + _swap_pairs(qh) * sinq
        s = jax.lax.dot_general(qh.astype(bf16), kkb,
                                (((1,), (1,)), ((), ())),
                                preferred_element_type=f32)
        s = jnp.maximum(s * SCALE, 0.0)
        return wb[:, h:h + 1] * s.astype(bf16).astype(f32)

    def group(b):
        t = [term(b + j) for j in range(8)]
        return ((t[0] + t[4]) + (t[2] + t[6])) + ((t[1] + t[5]) + (t[3] + t[7]))

    acc = ((group(0) + group(8)) + group(16)) + group(24)
    bend = jax.lax.broadcasted_iota(jnp.int32, acc.shape, 1) * RATIO + (RATIO - 1)
    mask = bend <= pos_ref[...]
    out_ref[...] = jnp.where(mask, acc, NEG)


def _k4_body(s_ref, out_ref):
    v = s_ref[...]
    n = v.shape[1]
    logn = int(math.log2(n))
    idx = jax.lax.broadcasted_iota(jnp.int32, v.shape, 1)
    lane = idx
    for k in range(logn):
        for j in range(k, -1, -1):
            s = 1 << j
            low = (lane & s) == 0
            pv = jnp.where(low, jnp.roll(v, -s, axis=1), jnp.roll(v, s, axis=1))
            pi = jnp.where(low, jnp.roll(idx, -s, axis=1), jnp.roll(idx, s, axis=1))
            desc = (lane & (2 << k)) == 0
            self_first = (v > pv) | ((v == pv) & (idx < pi))
            keep = self_first ^ (low != desc)
            v = jnp.where(keep, v, pv)
            idx = jnp.where(keep, idx, pi)
    out_ref[...] = idx[:, :TOPK]


def kernel(hidden_states, qr, positions, Wq_b, Wweights, Wkv, Wgate, ape, norm_w):
    inv_freq = jnp.asarray(
        1.0 / (10000.0 ** (np.arange(0, ROPE, 2, dtype=np.float32) / ROPE)))
    ang_q = positions.astype(f32)[:, None] * inv_freq[None, :]
    cos_q, sin_q = jnp.cos(ang_q), jnp.sin(ang_q)
    cos2q = jnp.concatenate(
        [jnp.ones((T, D - ROPE), f32),
         jnp.stack([cos_q, cos_q], -1).reshape(T, ROPE)], axis=1)
    sin2q = jnp.concatenate(
        [jnp.zeros((T, D - ROPE), f32),
         jnp.stack([-sin_q, sin_q], -1).reshape(T, ROPE)], axis=1)

    bpos = jnp.arange(BLOCKS, dtype=f32) * RATIO + (RATIO - 1)
    ang_k = bpos[:, None] * inv_freq[None, :]
    cos_k, sin_k = jnp.cos(ang_k), jnp.sin(ang_k)
    cos2k = jnp.concatenate(
        [jnp.ones((BLOCKS, D - ROPE), f32),
         jnp.stack([cos_k, cos_k], -1).reshape(BLOCKS, ROPE)], axis=1)
    sin2k = jnp.concatenate(
        [jnp.zeros((BLOCKS, D - ROPE), f32),
         jnp.stack([-sin_k, sin_k], -1).reshape(BLOCKS, ROPE)], axis=1)

    ape_tile = jnp.tile(ape, (TT // RATIO, 1))
    pos2d = positions.reshape(T, 1)
    normw2d = norm_w.reshape(1, D)
    nt = T // TT

    kvape, w = pl.pallas_call(
        _k1_body,
        grid=(nt,),
        in_specs=[
            pl.BlockSpec((TT, HIDDEN), lambda i: (i, 0)),
            pl.BlockSpec((HIDDEN, 2 * D), lambda i: (0, 0)),
            pl.BlockSpec((HIDDEN, 2 * D), lambda i: (0, 0)),
            pl.BlockSpec((HIDDEN, H), lambda i: (0, 0)),
            pl.BlockSpec((TT, 2 * D), lambda i: (0, 0)),
        ],
        out_specs=[
            pl.BlockSpec((TT, 2 * D), lambda i: (i, 0)),
            pl.BlockSpec((TT, H), lambda i: (i, 0)),
        ],
        out_shape=[
            jax.ShapeDtypeStruct((T, 2 * D), f32),
            jax.ShapeDtypeStruct((T, H), f32),
        ],
    )(hidden_states, Wkv, Wgate, Wweights, ape_tile)

    kv4 = kvape.reshape(BLOCKS, RATIO, 2 * D)
    rs = [kv4[:, j, :] for j in range(RATIO)]
    pre = pl.pallas_call(
        _k2_body,
        out_shape=jax.ShapeDtypeStruct((BLOCKS, D), f32),
    )(*rs)

    rstd = jax.lax.rsqrt(jnp.mean(pre.astype(f32) ** 2, axis=-1, keepdims=True)
                         + EPS)

    kk = pl.pallas_call(
        _k2b_body,
        out_shape=jax.ShapeDtypeStruct((BLOCKS, D), f32),
    )(pre, rstd, normw2d, cos2k, sin2k)

    idx_score = pl.pallas_call(
        _k3_body,
        grid=(T // TT3,),
        in_specs=[
            pl.BlockSpec((TT3, QR_RANK), lambda i: (i, 0)),
            pl.BlockSpec((QR_RANK, H * D), lambda i: (0, 0)),
            pl.BlockSpec((TT3, H), lambda i: (i, 0)),
            pl.BlockSpec((BLOCKS, D), lambda i: (0, 0)),
            pl.BlockSpec((TT3, D), lambda i: (i, 0)),
            pl.BlockSpec((TT3, D), lambda i: (i, 0)),
            pl.BlockSpec((TT3, 1), lambda i: (i, 0)),
        ],
        out_specs=pl.BlockSpec((TT3, BLOCKS), lambda i: (i, 0)),
        out_shape=jax.ShapeDtypeStruct((T, BLOCKS), f32),
    )(qr, Wq_b, w, kk, cos2q, sin2q, pos2d)

    topk_idx = pl.pallas_call(
        _k4_body,
        grid=(T // RT,),
        in_specs=[pl.BlockSpec((RT, BLOCKS), lambda i: (i, 0))],
        out_specs=pl.BlockSpec((RT, TOPK), lambda i: (i, 0)),
        out_shape=jax.ShapeDtypeStruct((T, TOPK), jnp.int32),
    )(idx_score)

    return idx_score, topk_idx


# speedup vs baseline: 1.3036x; 1.3036x over previous
import functools
import math

import jax
import jax.numpy as jnp
import numpy as np
from jax.experimental import pallas as pl

T = 4096
HIDDEN = 4096
QR_RANK = 1536
H = 32
D = 128
ROPE = 64
RATIO = 4
BLOCKS = T // RATIO
TOPK = 512
EPS = 1e-6
SCALE = D ** -0.5
NEG = -1e30

TT = 512
TT3 = 256
RT = 256

bf16 = jnp.bfloat16
f32 = jnp.float32


def _swap_pairs(x):
    lane = jax.lax.broadcasted_iota(jnp.int32, x.shape, len(x.shape) - 1)
    even = (lane & 1) == 0
    return jnp.where(even, jnp.roll(x, -1, axis=-1), jnp.roll(x, 1, axis=-1))


def _k1_body(h_ref, wkv_ref, wgate_ref, ww_ref, ape_ref, kvape_ref, w_ref):
    hb = h_ref[...].astype(bf16)
    kv = jnp.dot(hb, wkv_ref[...].astype(bf16), preferred_element_type=f32)
    gate_lin = jnp.dot(hb, wgate_ref[...].astype(bf16),
                       preferred_element_type=f32)
    gate = 1.0 / (1.0 + jnp.exp(-gate_lin))
    kvape_ref[...] = kv * gate + ape_ref[...]
    w_ref[...] = jnp.dot(hb, ww_ref[...].astype(bf16),
                         preferred_element_type=f32)


def _k2_body(r0_ref, r1_ref, r2_ref, r3_ref, pre_ref):
    refs = (r0_ref, r1_ref, r2_ref, r3_ref)
    t = []
    for j in range(RATIO):
        rj = refs[j][...]
        a = jnp.roll(rj[:, :D], 1, axis=0)
        row = jax.lax.broadcasted_iota(jnp.int32, a.shape, 0)
        t.append(jnp.where(row == 0, 0.0, a))
    for j in range(RATIO):
        t.append(refs[j][...][:, D:])
    y = [t[j] + t[j + 4] for j in range(4)]
    z = [y[j] + y[j + 2] for j in range(2)]
    pre_ref[...] = (z[0] + z[1]) * (1.0 / (2 * RATIO))


def _k2b_body(pre_ref, rstd_ref, normw_ref, cosk_ref, sink_ref, kk_ref):
    kk = (pre_ref[...] * rstd_ref[...]) * normw_ref[...]
    kk_ref[...] = kk * cosk_ref[...] + _swap_pairs(kk) * sink_ref[...]


def _k3_body(qr_ref, wqb_ref, w_ref, kk_ref, cosq_ref, sinq_ref, pos_ref,
             out_ref):
    q = jnp.dot(qr_ref[...].astype(bf16), wqb_ref[...].astype(bf16),
                preferred_element_type=f32)
    cosq = cosq_ref[...]
    sinq = sinq_ref[...]
    kkb = kk_ref[...].astype(bf16)
    wb = w_ref[...].astype(bf16).astype(f32)

    def term(h):
        qh = q[:, h * D:(h + 1) * D]
        qh = qh * cosq + _swap_pairs(qh) * sinq
        s = jax.lax.dot_general(qh.astype(bf16), kkb,
                                (((1,), (1,)), ((), ())),
                                preferred_element_type=f32)
        s = jnp.maximum(s * SCALE, 0.0)
        return wb[:, h:h + 1] * s.astype(bf16).astype(f32)

    def group(b):
        t = [term(b + j) for j in range(8)]
        return ((t[0] + t[4]) + (t[2] + t[6])) + ((t[1] + t[5]) + (t[3] + t[7]))

    acc = ((group(0) + group(8)) + group(16)) + group(24)
    bend = jax.lax.broadcasted_iota(jnp.int32, acc.shape, 1) * RATIO + (RATIO - 1)
    mask = bend <= pos_ref[...]
    out_ref[...] = jnp.where(mask, acc, NEG)


def _k4_body(s_ref, out_ref):
    v = s_ref[...]
    n = v.shape[1]
    logn = int(math.log2(n))
    idx = jax.lax.broadcasted_iota(jnp.int32, v.shape, 1)
    lane = idx
    for k in range(logn):
        for j in range(k, -1, -1):
            s = 1 << j
            low = (lane & s) == 0
            pv = jnp.where(low, jnp.roll(v, -s, axis=1), jnp.roll(v, s, axis=1))
            pi = jnp.where(low, jnp.roll(idx, -s, axis=1), jnp.roll(idx, s, axis=1))
            desc = (lane & (2 << k)) == 0
            self_first = (v > pv) | ((v == pv) & (idx < pi))
            keep = self_first ^ (low != desc)
            v = jnp.where(keep, v, pv)
            idx = jnp.where(keep, idx, pi)
    out_ref[...] = idx[:, :TOPK]


def kernel(hidden_states, qr, positions, Wq_b, Wweights, Wkv, Wgate, ape, norm_w):
    inv_freq = jnp.asarray(
        1.0 / (10000.0 ** (np.arange(0, ROPE, 2, dtype=np.float32) / ROPE)))
    ang_q = positions.astype(f32)[:, None] * inv_freq[None, :]
    cos_q, sin_q = jnp.cos(ang_q), jnp.sin(ang_q)
    cos2q = jnp.concatenate(
        [jnp.ones((T, D - ROPE), f32),
         jnp.stack([cos_q, cos_q], -1).reshape(T, ROPE)], axis=1)
    sin2q = jnp.concatenate(
        [jnp.zeros((T, D - ROPE), f32),
         jnp.stack([-sin_q, sin_q], -1).reshape(T, ROPE)], axis=1)

    bpos = jnp.arange(BLOCKS, dtype=f32) * RATIO + (RATIO - 1)
    ang_k = bpos[:, None] * inv_freq[None, :]
    cos_k, sin_k = jnp.cos(ang_k), jnp.sin(ang_k)
    cos2k = jnp.concatenate(
        [jnp.ones((BLOCKS, D - ROPE), f32),
         jnp.stack([cos_k, cos_k], -1).reshape(BLOCKS, ROPE)], axis=1)
    sin2k = jnp.concatenate(
        [jnp.zeros((BLOCKS, D - ROPE), f32),
         jnp.stack([-sin_k, sin_k], -1).reshape(BLOCKS, ROPE)], axis=1)

    ape_tile = jnp.tile(ape, (TT // RATIO, 1))
    pos2d = positions.reshape(T, 1)
    normw2d = norm_w.reshape(1, D)
    nt = T // TT

    kvape, w = pl.pallas_call(
        _k1_body,
        grid=(nt,),
        in_specs=[
            pl.BlockSpec((TT, HIDDEN), lambda i: (i, 0)),
            pl.BlockSpec((HIDDEN, 2 * D), lambda i: (0, 0)),
            pl.BlockSpec((HIDDEN, 2 * D), lambda i: (0, 0)),
            pl.BlockSpec((HIDDEN, H), lambda i: (0, 0)),
            pl.BlockSpec((TT, 2 * D), lambda i: (0, 0)),
        ],
        out_specs=[
            pl.BlockSpec((TT, 2 * D), lambda i: (i, 0)),
            pl.BlockSpec((TT, H), lambda i: (i, 0)),
        ],
        out_shape=[
            jax.ShapeDtypeStruct((T, 2 * D), f32),
            jax.ShapeDtypeStruct((T, H), f32),
        ],
    )(hidden_states, Wkv, Wgate, Wweights, ape_tile)

    kv4 = kvape.reshape(BLOCKS, RATIO, 2 * D)
    rs = [kv4[:, j, :] for j in range(RATIO)]
    pre = pl.pallas_call(
        _k2_body,
        out_shape=jax.ShapeDtypeStruct((BLOCKS, D), f32),
    )(*rs)

    rstd = jax.lax.rsqrt(jnp.mean(pre.astype(f32) ** 2, axis=-1, keepdims=True)
                         + EPS)

    kk = pl.pallas_call(
        _k2b_body,
        out_shape=jax.ShapeDtypeStruct((BLOCKS, D), f32),
    )(pre, rstd, normw2d, cos2k, sin2k)

    idx_score = pl.pallas_call(
        _k3_body,
        grid=(T // TT3,),
        in_specs=[
            pl.BlockSpec((TT3, QR_RANK), lambda i: (i, 0)),
            pl.BlockSpec((QR_RANK, H * D), lambda i: (0, 0)),
            pl.BlockSpec((TT3, H), lambda i: (i, 0)),
            pl.BlockSpec((BLOCKS, D), lambda i: (0, 0)),
            pl.BlockSpec((TT3, D), lambda i: (i, 0)),
            pl.BlockSpec((TT3, D), lambda i: (i, 0)),
            pl.BlockSpec((TT3, 1), lambda i: (i, 0)),
        ],
        out_specs=pl.BlockSpec((TT3, BLOCKS), lambda i: (i, 0)),
        out_shape=jax.ShapeDtypeStruct((T, BLOCKS), f32),
    )(qr, Wq_b, w, kk, cos2q, sin2q, pos2d)

    half = T // 2
    topk_lo = pl.pallas_call(
        _k4_body,
        grid=(half // RT,),
        in_specs=[pl.BlockSpec((RT, TOPK), lambda i: (i, 0))],
        out_specs=pl.BlockSpec((RT, TOPK), lambda i: (i, 0)),
        out_shape=jax.ShapeDtypeStruct((half, TOPK), jnp.int32),
    )(jax.lax.slice(idx_score, (0, 0), (half, TOPK)))
    topk_hi = pl.pallas_call(
        _k4_body,
        grid=(half // RT,),
        in_specs=[pl.BlockSpec((RT, BLOCKS), lambda i: (i, 0))],
        out_specs=pl.BlockSpec((RT, TOPK), lambda i: (i, 0)),
        out_shape=jax.ShapeDtypeStruct((half, TOPK), jnp.int32),
    )(jax.lax.slice(idx_score, (half, 0), (T, BLOCKS)))
    topk_idx = jnp.concatenate([topk_lo, topk_hi], axis=0)

    return idx_score, topk_idx
